# baseline (speedup 1.0000x reference)
import functools

import jax
import jax.numpy as jnp
from jax import lax
from jax.experimental import pallas as pl
from jax.experimental.pallas import tpu as pltpu
from jax.experimental.pallas import tpu_sc as plsc

N = 10000
E = 160000
F = 256
H = 512

NSUB = 16
NCHUNK = 125
CW = 80
CWC = 128
NCHC = 80
EP = NSUB * NCHC * CWC
TRASH = 10200
NPAD = 10240
RPT = NPAD // NSUB
BN = 400
GRID = N // BN
BND = 2000
GRIDD = N // BND

_mesh = plsc.VectorSubcoreMesh(core_axis_name="c", subcore_axis_name="s")


def _zero_vec(ref, nwords):
    def body(k, carry):
        ref[pl.ds(k * 16, 16)] = jnp.zeros((16,), jnp.float32)
        return carry
    lax.fori_loop(0, nwords // 16, body, 0)


@functools.partial(
    pl.kernel,
    out_type=jax.ShapeDtypeStruct((2, NPAD), jnp.float32),
    mesh=_mesh,
    scratch_types=[
        pltpu.VMEM((NCHUNK, CW), jnp.int32),
        pltpu.VMEM((CW,), jnp.float32),
        pltpu.VMEM((RPT,), jnp.float32),
        pltpu.VMEM_SHARED((NPAD,), jnp.float32),
    ],
)
def _degrees(src3, dst3, deg2, idx_v, ones_v, buf, acc):
    c = lax.axis_index("c")
    s = lax.axis_index("s")

    @pl.when(c == 0)
    def _():
        pltpu.sync_copy(src3.at[s], idx_v)

    @pl.when(c == 1)
    def _():
        pltpu.sync_copy(dst3.at[s], idx_v)
    for l in range(CW // 16):
        ones_v[pl.ds(l * 16, 16)] = jnp.ones((16,), jnp.float32)
    _zero_vec(buf, RPT)
    pltpu.sync_copy(buf, acc.at[pl.ds(s * RPT, RPT)])
    plsc.subcore_barrier()

    def body(j, carry):
        pltpu.sync_copy(ones_v, acc.at[idx_v.at[j]], add=True)
        return carry
    lax.fori_loop(0, NCHUNK, body, 0)
    plsc.subcore_barrier()
    pltpu.sync_copy(acc.at[pl.ds(s * RPT, RPT)], buf)
    pltpu.sync_copy(buf, deg2.at[c, pl.ds(s * RPT, RPT)])


QW = 64
NQ = F // QW


def _normalize_body(feat, dego, degi, y, nsrc, ndst):
    ns = lax.rsqrt(jnp.maximum(dego[...], 1.0))
    nd = lax.rsqrt(jnp.maximum(degi[...], 1.0))
    y[...] = feat[...] * ns
    nsrc[...] = ns
    ndst[...] = nd


def _normalize(features, deg_out, deg_in):
    return pl.pallas_call(
        _normalize_body,
        grid=(GRID,),
        in_specs=[
            pl.BlockSpec((BN, F), lambda i: (i, 0)),
            pl.BlockSpec((BN, 1), lambda i: (i, 0)),
            pl.BlockSpec((BN, 1), lambda i: (i, 0)),
        ],
        out_specs=[
            pl.BlockSpec((BN, F), lambda i: (i, 0)),
            pl.BlockSpec((BN, 1), lambda i: (i, 0)),
            pl.BlockSpec((BN, 1), lambda i: (i, 0)),
        ],
        out_shape=[
            jax.ShapeDtypeStruct((N, F), jnp.float32),
            jax.ShapeDtypeStruct((N, 1), jnp.float32),
            jax.ShapeDtypeStruct((N, 1), jnp.float32),
        ],
    )(features, deg_out, deg_in)


KSUP = 4

NSUP = NCHC // KSUP

@functools.partial(
    pl.kernel,
    out_type=jax.ShapeDtypeStruct((NPAD, F), jnp.float32),
    mesh=_mesh,
    scratch_types=[
        pltpu.VMEM((NCHC, CWC), jnp.int32),
        pltpu.VMEM((NCHC, CWC), jnp.int32),
        pltpu.VMEM((2, KSUP, CWC, QW), jnp.float32),
        pltpu.VMEM((16, QW), jnp.float32),
        pltpu.VMEM_SHARED((NPAD, QW), jnp.float32),
        pltpu.SemaphoreType.DMA,
        pltpu.SemaphoreType.DMA,
    ],
    compiler_params=pltpu.CompilerParams(use_tc_tiling_on_sc=False),
)
def _aggregate(yc, s3, d3, agg, src_v, dst_v, st, zb, acc, gsem, ssem):
    c = lax.axis_index("c")
    s = lax.axis_index("s")
    pltpu.sync_copy(d3.at[s], dst_v)
    pltpu.sync_copy(s3.at[s], src_v)

    def zrow(r, carry):
        for l in range(QW // 16):
            zb[r, pl.ds(l * 16, 16)] = jnp.zeros((16,), jnp.float32)
        return carry
    lax.fori_loop(0, 16, zrow, 0)

    def fire(t, buf, tab):
        base = t * KSUP
        for b in range(KSUP):
            pltpu.async_copy(
                tab.at[src_v.at[base + b]], st.at[buf, b], gsem)

    def drain(buf):
        for b in range(KSUP):
            pltpu.make_async_copy(
                yc.at[0, pl.ds(0, CWC)], st.at[buf, b], gsem).wait()

    def fire_sc(t, buf):
        base = t * KSUP
        for b in range(KSUP):
            pltpu.async_copy(
                st.at[buf, b], acc.at[dst_v.at[base + b]], ssem, add=True)

    def drain_sc(buf):
        for b in range(KSUP):
            pltpu.make_async_copy(
                yc.at[0, pl.ds(0, CWC)], st.at[buf, b], ssem).wait()

    for p in range(2):
        q = 2 * c + p
        tab = yc.at[q]

        def zcp(t, carry):
            pltpu.sync_copy(zb, acc.at[pl.ds(s * RPT + t * 16, 16)])
            return carry
        lax.fori_loop(0, RPT // 16, zcp, 0)
        plsc.subcore_barrier()

        fire(0, 0, tab)

        def super_step(t, carry):
            pp = lax.rem(t, 2)
            drain(pp)

            @pl.when(t >= 1)
            def _():
                drain_sc(1 - pp)

            @pl.when(t + 1 < NSUP)
            def _():
                fire(t + 1, 1 - pp, tab)
            fire_sc(t, pp)
            return carry
        lax.fori_loop(0, NSUP, super_step, 0)
        drain_sc(lax.rem(NSUP - 1, 2))
        plsc.subcore_barrier()

        def out_cp(t, carry):
            pltpu.sync_copy(acc.at[pl.ds(s * RPT + t * 16, 16)], zb)
            pltpu.sync_copy(
                zb, agg.at[pl.ds(s * RPT + t * 16, 16), pl.ds(q * QW, QW)])
            return carry
        lax.fori_loop(0, RPT // 16, out_cp, 0)
        if p == 0:
            lax.fori_loop(0, 16, zrow, 0)
            plsc.subcore_barrier()


def _dense_body(a, nd, ns, w1, b1, w2, g):
    h = lax.dot_general(
        a[...], w1[...], (((1,), (0,)), ((), ())),
        precision=lax.Precision.HIGHEST, preferred_element_type=jnp.float32)
    h = h * nd[...] + b1[...]
    h = jnp.maximum(h, 0.0) * ns[...]
    g[...] = lax.dot_general(
        h, w2[...], (((1,), (0,)), ((), ())),
        precision=lax.Precision.HIGHEST, preferred_element_type=jnp.float32)


def _dense(agg, ndst, nsrc, W1, b1, W2):
    return pl.pallas_call(
        _dense_body,
        grid=(GRIDD,),
        in_specs=[
            pl.BlockSpec((BND, F), lambda i: (i, 0)),
            pl.BlockSpec((BND, 1), lambda i: (i, 0)),
            pl.BlockSpec((BND, 1), lambda i: (i, 0)),
            pl.BlockSpec((F, H), lambda i: (0, 0)),
            pl.BlockSpec((1, H), lambda i: (0, 0)),
            pl.BlockSpec((H, 1), lambda i: (0, 0)),
        ],
        out_specs=pl.BlockSpec((BND, 1), lambda i: (i, 0)),
        out_shape=jax.ShapeDtypeStruct((N, 1), jnp.float32),
    )(agg, ndst, nsrc, W1, b1, W2)


@functools.partial(
    pl.kernel,
    out_type=jax.ShapeDtypeStruct((2, NPAD), jnp.float32),
    mesh=_mesh,
    scratch_types=[
        pltpu.VMEM((NCHC, CWC), jnp.int32),
        pltpu.VMEM((NCHC, CWC), jnp.int32),
        pltpu.VMEM((NPAD,), jnp.float32),
        pltpu.VMEM((NPAD,), jnp.float32),
        pltpu.VMEM((NPAD // 32,), jnp.float32),
        pltpu.VMEM((NPAD // 32,), jnp.float32),
        pltpu.VMEM((NPAD // 32,), jnp.float32),
        pltpu.VMEM((16,), jnp.float32),
        pltpu.VMEM_SHARED((NSUB, NPAD // 2), jnp.float32),
    ],
    compiler_params=pltpu.CompilerParams(
        use_tc_tiling_on_sc=False, needs_layout_passes=False),
)
def _layer2(g1, s3, d3, ndp, b2h, o2, src_v, dst_v, gtab, accl,
            lbuf, nbuf, obuf, b2v, slots):
    c = lax.axis_index("c")
    s = lax.axis_index("s")
    pltpu.sync_copy(s3.at[s], src_v)
    pltpu.sync_copy(d3.at[s], dst_v)
    pltpu.sync_copy(g1, gtab.at[pl.ds(0, N)])
    pltpu.sync_copy(b2h, b2v)
    _zero_vec(accl, NPAD)

    def edge(j, carry):
        for k in range(CWC // 16):
            sl = pl.ds(k * 16, 16)
            si = src_v[j, sl]
            di = dst_v[j, sl]
            vals = plsc.load_gather(gtab, [si])
            plsc.addupdate_scatter(accl, [di], vals)
        return carry
    lax.fori_loop(0, NCHC, edge, 0)

    HN = NPAD // 2
    HRPT = NPAD // 32
    b2r = b2v[...]
    for r in range(2):
        pltpu.sync_copy(accl.at[pl.ds(r * HN, HN)], slots.at[s])
        plsc.subcore_barrier()
        off = s * HRPT
        glob = r * HN + off
        pltpu.sync_copy(slots.at[0, pl.ds(off, HRPT)], lbuf)
        for t in range(1, NSUB):
            pltpu.sync_copy(slots.at[t, pl.ds(off, HRPT)], nbuf)

            def accrow(k, carry):
                sl = pl.ds(k * 16, 16)
                lbuf[sl] = lbuf[sl] + nbuf[sl]
                return carry
            lax.fori_loop(0, HRPT // 16, accrow, 0)
        pltpu.sync_copy(ndp.at[pl.ds(glob, HRPT)], nbuf)

        def scale(k, carry):
            sl = pl.ds(k * 16, 16)
            obuf[sl] = lbuf[sl] * nbuf[sl] + b2r
            return carry
        lax.fori_loop(0, HRPT // 16, scale, 0)
        pltpu.sync_copy(obuf, o2.at[c, pl.ds(glob, HRPT)])
        plsc.subcore_barrier()


def kernel(features, edge_index, W1, b1, W2, b2):
    src = edge_index[0].astype(jnp.int32)
    dst = edge_index[1].astype(jnp.int32)
    src3 = src.reshape(NSUB, NCHUNK, CW)
    dst3 = dst.reshape(NSUB, NCHUNK, CW)
    srcp = jnp.concatenate(
        [src, jnp.zeros((EP - E,), jnp.int32)]).reshape(NSUB, NCHC, CWC)
    dstp = jnp.concatenate(
        [dst, jnp.full((EP - E,), TRASH, jnp.int32)]).reshape(NSUB, NCHC, CWC)

    deg2 = _degrees(src3, dst3)
    deg_out = deg2[0, :N, None]
    deg_in = deg2[1, :N, None]

    y, nsrc, ndst = _normalize(features, deg_out, deg_in)
    ycat = y.reshape(N, NQ, QW).transpose(1, 0, 2)

    agg = _aggregate(ycat, srcp, dstp)
    g = _dense(agg, ndst, nsrc, W1, b1.reshape(1, H), W2)

    ndp = jnp.concatenate([ndst[:, 0], jnp.zeros((NPAD - N,), jnp.float32)])
    b2h = jnp.broadcast_to(b2, (16,))
    o2 = _layer2(g[:, 0], srcp, dstp, ndp, b2h)
    return o2[0, :N].reshape(N, 1)


# speedup vs baseline: 1.6375x; 1.6375x over previous
import functools

import jax
import jax.numpy as jnp
from jax import lax
from jax.experimental import pallas as pl
from jax.experimental.pallas import tpu as pltpu
from jax.experimental.pallas import tpu_sc as plsc

N = 10000
E = 160000
F = 256
H = 512

NSUB = 16
NCHUNK = 125
CW = 80
CWC = 128
NCHC = 80
EP = NSUB * NCHC * CWC
TRASH = 10200
NPAD = 10240
RPT = NPAD // NSUB
BN = 400
GRID = N // BN
BND = 2000
GRIDD = N // BND

_mesh = plsc.VectorSubcoreMesh(core_axis_name="c", subcore_axis_name="s")


def _zero_vec(ref, nwords):
    def body(k, carry):
        ref[pl.ds(k * 16, 16)] = jnp.zeros((16,), jnp.float32)
        return carry
    lax.fori_loop(0, nwords // 16, body, 0)


@functools.partial(
    pl.kernel,
    out_type=jax.ShapeDtypeStruct((2, NPAD), jnp.float32),
    mesh=_mesh,
    scratch_types=[
        pltpu.VMEM((NCHUNK, CW), jnp.int32),
        pltpu.VMEM((CW,), jnp.float32),
        pltpu.VMEM((RPT,), jnp.float32),
        pltpu.VMEM_SHARED((NPAD,), jnp.float32),
    ],
)
def _degrees(src3, dst3, deg2, idx_v, ones_v, buf, acc):
    c = lax.axis_index("c")
    s = lax.axis_index("s")

    @pl.when(c == 0)
    def _():
        pltpu.sync_copy(src3.at[s], idx_v)

    @pl.when(c == 1)
    def _():
        pltpu.sync_copy(dst3.at[s], idx_v)
    for l in range(CW // 16):
        ones_v[pl.ds(l * 16, 16)] = jnp.ones((16,), jnp.float32)
    _zero_vec(buf, RPT)
    pltpu.sync_copy(buf, acc.at[pl.ds(s * RPT, RPT)])
    plsc.subcore_barrier()

    def body(j, carry):
        pltpu.sync_copy(ones_v, acc.at[idx_v.at[j]], add=True)
        return carry
    lax.fori_loop(0, NCHUNK, body, 0)
    plsc.subcore_barrier()
    pltpu.sync_copy(acc.at[pl.ds(s * RPT, RPT)], buf)
    pltpu.sync_copy(buf, deg2.at[c, pl.ds(s * RPT, RPT)])


QW = 64
NQ = F // QW


def _normalize_body(feat, dego, degi, y, nsrc, ndst):
    ns = lax.rsqrt(jnp.maximum(dego[...], 1.0))
    nd = lax.rsqrt(jnp.maximum(degi[...], 1.0))
    y[...] = feat[...] * ns
    nsrc[...] = ns
    ndst[...] = nd


def _normalize(features, deg_out, deg_in):
    return pl.pallas_call(
        _normalize_body,
        grid=(GRID,),
        in_specs=[
            pl.BlockSpec((BN, F), lambda i: (i, 0)),
            pl.BlockSpec((BN, 1), lambda i: (i, 0)),
            pl.BlockSpec((BN, 1), lambda i: (i, 0)),
        ],
        out_specs=[
            pl.BlockSpec((BN, F), lambda i: (i, 0)),
            pl.BlockSpec((BN, 1), lambda i: (i, 0)),
            pl.BlockSpec((BN, 1), lambda i: (i, 0)),
        ],
        out_shape=[
            jax.ShapeDtypeStruct((N, F), jnp.float32),
            jax.ShapeDtypeStruct((N, 1), jnp.float32),
            jax.ShapeDtypeStruct((N, 1), jnp.float32),
        ],
    )(features, deg_out, deg_in)


KSUP = 5

NSUP = NCHUNK // KSUP

@functools.partial(
    pl.kernel,
    out_type=jax.ShapeDtypeStruct((NPAD, F), jnp.float32),
    mesh=_mesh,
    scratch_types=[
        pltpu.VMEM((NCHUNK, CW), jnp.int32),
        pltpu.VMEM((NCHUNK, CW), jnp.int32),
        pltpu.VMEM((2, KSUP, CW, QW), jnp.float32),
        pltpu.VMEM((16, QW), jnp.float32),
        pltpu.VMEM_SHARED((NPAD, QW), jnp.float32),
        pltpu.SemaphoreType.DMA,
        pltpu.SemaphoreType.DMA,
    ],
    compiler_params=pltpu.CompilerParams(use_tc_tiling_on_sc=False),
)
def _aggregate(yc, s3, d3, agg, src_v, dst_v, st, zb, acc, gsem, ssem):
    c = lax.axis_index("c")
    s = lax.axis_index("s")
    pltpu.sync_copy(d3.at[s], dst_v)
    pltpu.sync_copy(s3.at[s], src_v)

    def zrow(r, carry):
        for l in range(QW // 16):
            zb[r, pl.ds(l * 16, 16)] = jnp.zeros((16,), jnp.float32)
        return carry
    lax.fori_loop(0, 16, zrow, 0)

    def fire(t, buf, tab):
        base = t * KSUP
        for b in range(KSUP):
            pltpu.async_copy(
                tab.at[src_v.at[base + b]], st.at[buf, b], gsem)

    def drain(buf):
        for b in range(KSUP):
            pltpu.make_async_copy(
                yc.at[0, pl.ds(0, CW)], st.at[buf, b], gsem).wait()

    def fire_sc(t, buf):
        base = t * KSUP
        for b in range(KSUP):
            pltpu.async_copy(
                st.at[buf, b], acc.at[dst_v.at[base + b]], ssem, add=True)

    def drain_sc(buf):
        for b in range(KSUP):
            pltpu.make_async_copy(
                yc.at[0, pl.ds(0, CW)], st.at[buf, b], ssem).wait()

    for p in range(2):
        q = 2 * c + p
        tab = yc.at[q]

        def zcp(t, carry):
            pltpu.sync_copy(zb, acc.at[pl.ds(s * RPT + t * 16, 16)])
            return carry
        lax.fori_loop(0, RPT // 16, zcp, 0)
        plsc.subcore_barrier()

        fire(0, 0, tab)

        def super_step(t, carry):
            pp = lax.rem(t, 2)
            drain(pp)

            @pl.when(t >= 1)
            def _():
                drain_sc(1 - pp)

            @pl.when(t + 1 < NSUP)
            def _():
                fire(t + 1, 1 - pp, tab)
            fire_sc(t, pp)
            return carry
        lax.fori_loop(0, NSUP, super_step, 0)
        drain_sc(lax.rem(NSUP - 1, 2))
        plsc.subcore_barrier()

        def out_cp(t, carry):
            pltpu.sync_copy(acc.at[pl.ds(s * RPT + t * 16, 16)], zb)
            pltpu.sync_copy(
                zb, agg.at[pl.ds(s * RPT + t * 16, 16), pl.ds(q * QW, QW)])
            return carry
        lax.fori_loop(0, RPT // 16, out_cp, 0)
        if p == 0:
            lax.fori_loop(0, 16, zrow, 0)
            plsc.subcore_barrier()


def _dense_body(a, nd, ns, w1, b1, w2, g):
    h = lax.dot_general(
        a[...], w1[...], (((1,), (0,)), ((), ())),
        precision=lax.Precision.HIGHEST, preferred_element_type=jnp.float32)
    h = h * nd[...] + b1[...]
    h = jnp.maximum(h, 0.0) * ns[...]
    g[...] = lax.dot_general(
        h, w2[...], (((1,), (0,)), ((), ())),
        precision=lax.Precision.HIGHEST, preferred_element_type=jnp.float32)


def _dense(agg, ndst, nsrc, W1, b1, W2):
    return pl.pallas_call(
        _dense_body,
        grid=(GRIDD,),
        in_specs=[
            pl.BlockSpec((BND, F), lambda i: (i, 0)),
            pl.BlockSpec((BND, 1), lambda i: (i, 0)),
            pl.BlockSpec((BND, 1), lambda i: (i, 0)),
            pl.BlockSpec((F, H), lambda i: (0, 0)),
            pl.BlockSpec((1, H), lambda i: (0, 0)),
            pl.BlockSpec((H, 1), lambda i: (0, 0)),
        ],
        out_specs=pl.BlockSpec((BND, 1), lambda i: (i, 0)),
        out_shape=jax.ShapeDtypeStruct((N, 1), jnp.float32),
    )(agg, ndst, nsrc, W1, b1, W2)


@functools.partial(
    pl.kernel,
    out_type=jax.ShapeDtypeStruct((2, NPAD), jnp.float32),
    mesh=_mesh,
    scratch_types=[
        pltpu.VMEM((NCHUNK, CW), jnp.int32),
        pltpu.VMEM((NCHUNK, CW), jnp.int32),
        pltpu.VMEM((NPAD,), jnp.float32),
        pltpu.VMEM((NPAD,), jnp.float32),
        pltpu.VMEM((NPAD // 32,), jnp.float32),
        pltpu.VMEM((NPAD // 32,), jnp.float32),
        pltpu.VMEM((NPAD // 32,), jnp.float32),
        pltpu.VMEM((16,), jnp.float32),
        pltpu.VMEM_SHARED((NSUB, NPAD // 2), jnp.float32),
    ],
    compiler_params=pltpu.CompilerParams(
        use_tc_tiling_on_sc=False, needs_layout_passes=False),
)
def _layer2(g1, s3, d3, ndp, b2h, o2, src_v, dst_v, gtab, accl,
            lbuf, nbuf, obuf, b2v, slots):
    c = lax.axis_index("c")
    s = lax.axis_index("s")
    pltpu.sync_copy(s3.at[s], src_v)
    pltpu.sync_copy(d3.at[s], dst_v)
    pltpu.sync_copy(g1, gtab.at[pl.ds(0, N)])
    pltpu.sync_copy(b2h, b2v)
    _zero_vec(accl, NPAD)

    def edge(j, carry):
        for k in range(CW // 16):
            sl = pl.ds(k * 16, 16)
            si = src_v[j, sl]
            di = dst_v[j, sl]
            vals = plsc.load_gather(gtab, [si])
            plsc.addupdate_scatter(accl, [di], vals)
        return carry
    lax.fori_loop(0, NCHUNK, edge, 0)

    HN = NPAD // 2
    HRPT = NPAD // 32
    b2r = b2v[...]
    for r in range(2):
        pltpu.sync_copy(accl.at[pl.ds(r * HN, HN)], slots.at[s])
        plsc.subcore_barrier()
        off = s * HRPT
        glob = r * HN + off
        pltpu.sync_copy(slots.at[0, pl.ds(off, HRPT)], lbuf)
        for t in range(1, NSUB):
            pltpu.sync_copy(slots.at[t, pl.ds(off, HRPT)], nbuf)

            def accrow(k, carry):
                sl = pl.ds(k * 16, 16)
                lbuf[sl] = lbuf[sl] + nbuf[sl]
                return carry
            lax.fori_loop(0, HRPT // 16, accrow, 0)
        pltpu.sync_copy(ndp.at[pl.ds(glob, HRPT)], nbuf)

        def scale(k, carry):
            sl = pl.ds(k * 16, 16)
            obuf[sl] = lbuf[sl] * nbuf[sl] + b2r
            return carry
        lax.fori_loop(0, HRPT // 16, scale, 0)
        pltpu.sync_copy(obuf, o2.at[c, pl.ds(glob, HRPT)])
        plsc.subcore_barrier()


def kernel(features, edge_index, W1, b1, W2, b2):
    src = edge_index[0].astype(jnp.int32)
    dst = edge_index[1].astype(jnp.int32)
    src3 = src.reshape(NSUB, NCHUNK, CW)
    dst3 = dst.reshape(NSUB, NCHUNK, CW)

    deg2 = _degrees(src3, dst3)
    deg_out = deg2[0, :N, None]
    deg_in = deg2[1, :N, None]

    y, nsrc, ndst = _normalize(features, deg_out, deg_in)
    ycat = y.reshape(N, NQ, QW).transpose(1, 0, 2)

    agg = _aggregate(ycat, src3, dst3)
    g = _dense(agg, ndst, nsrc, W1, b1.reshape(1, H), W2)

    ndp = jnp.concatenate([ndst[:, 0], jnp.zeros((NPAD - N,), jnp.float32)])
    b2h = jnp.broadcast_to(b2, (16,))
    o2 = _layer2(g[:, 0], src3, dst3, ndp, b2h)
    return o2[0, :N].reshape(N, 1)


# speedup vs baseline: 1.8678x; 1.1406x over previous
import functools

import jax
import jax.numpy as jnp
from jax import lax
from jax.experimental import pallas as pl
from jax.experimental.pallas import tpu as pltpu
from jax.experimental.pallas import tpu_sc as plsc

N = 10000
E = 160000
F = 256
H = 512

NSUB = 16
NCHUNK = 125
CW = 80
CWC = 128
NCHC = 80
EP = NSUB * NCHC * CWC
TRASH = 10200
NPAD = 10240
RPT = NPAD // NSUB
BN = 400
GRID = N // BN
BND = 2000
GRIDD = N // BND

_mesh = plsc.VectorSubcoreMesh(core_axis_name="c", subcore_axis_name="s")


def _zero_vec(ref, nwords):
    def body(k, carry):
        ref[pl.ds(k * 16, 16)] = jnp.zeros((16,), jnp.float32)
        return carry
    lax.fori_loop(0, nwords // 16, body, 0)


@functools.partial(
    pl.kernel,
    out_type=jax.ShapeDtypeStruct((2, NPAD), jnp.float32),
    mesh=_mesh,
    scratch_types=[
        pltpu.VMEM((NCHUNK, CW), jnp.int32),
        pltpu.VMEM((CW,), jnp.float32),
        pltpu.VMEM((RPT,), jnp.float32),
        pltpu.VMEM_SHARED((NPAD,), jnp.float32),
    ],
)
def _degrees(src3, dst3, deg2, idx_v, ones_v, buf, acc):
    c = lax.axis_index("c")
    s = lax.axis_index("s")

    @pl.when(c == 0)
    def _():
        pltpu.sync_copy(src3.at[s], idx_v)

    @pl.when(c == 1)
    def _():
        pltpu.sync_copy(dst3.at[s], idx_v)
    for l in range(CW // 16):
        ones_v[pl.ds(l * 16, 16)] = jnp.ones((16,), jnp.float32)
    _zero_vec(buf, RPT)
    pltpu.sync_copy(buf, acc.at[pl.ds(s * RPT, RPT)])
    plsc.subcore_barrier()

    def body(j, carry):
        pltpu.sync_copy(ones_v, acc.at[idx_v.at[j]], add=True)
        return carry
    lax.fori_loop(0, NCHUNK, body, 0)
    plsc.subcore_barrier()
    pltpu.sync_copy(acc.at[pl.ds(s * RPT, RPT)], buf)
    pltpu.sync_copy(buf, deg2.at[c, pl.ds(s * RPT, RPT)])


QW = 64
NQ = F // QW


def _normalize_body(feat, dego, degi, y, nsrc, ndst):
    ns = lax.rsqrt(jnp.maximum(dego[...], 1.0))
    nd = lax.rsqrt(jnp.maximum(degi[...], 1.0))
    y[...] = feat[...] * ns
    nsrc[...] = ns
    ndst[...] = nd


def _normalize(features, deg_out, deg_in):
    return pl.pallas_call(
        _normalize_body,
        grid=(GRID,),
        in_specs=[
            pl.BlockSpec((BN, F), lambda i: (i, 0)),
            pl.BlockSpec((BN, 1), lambda i: (i, 0)),
            pl.BlockSpec((BN, 1), lambda i: (i, 0)),
        ],
        out_specs=[
            pl.BlockSpec((BN, F), lambda i: (i, 0)),
            pl.BlockSpec((BN, 1), lambda i: (i, 0)),
            pl.BlockSpec((BN, 1), lambda i: (i, 0)),
        ],
        out_shape=[
            jax.ShapeDtypeStruct((N, F), jnp.float32),
            jax.ShapeDtypeStruct((N, 1), jnp.float32),
            jax.ShapeDtypeStruct((N, 1), jnp.float32),
        ],
    )(features, deg_out, deg_in)


KSUP = 5

NSUP = NCHUNK // KSUP

@functools.partial(
    pl.kernel,
    out_type=jax.ShapeDtypeStruct((NPAD, F), jnp.float32),
    mesh=_mesh,
    scratch_types=[
        pltpu.VMEM((NCHUNK, CW), jnp.int32),
        pltpu.VMEM((NCHUNK, CW), jnp.int32),
        pltpu.VMEM((2, KSUP, CW, QW), jnp.float32),
        pltpu.VMEM((128, QW), jnp.float32),
        pltpu.VMEM_SHARED((NPAD, QW), jnp.float32),
        pltpu.SemaphoreType.DMA,
        pltpu.SemaphoreType.DMA,
    ],
    compiler_params=pltpu.CompilerParams(use_tc_tiling_on_sc=False),
)
def _aggregate(yc, s3, d3, agg, src_v, dst_v, st, zb, acc, gsem, ssem):
    c = lax.axis_index("c")
    s = lax.axis_index("s")
    pltpu.sync_copy(d3.at[s], dst_v)
    pltpu.sync_copy(s3.at[s], src_v)

    def zrow(r, carry):
        for l in range(QW // 16):
            zb[r, pl.ds(l * 16, 16)] = jnp.zeros((16,), jnp.float32)
        return carry
    lax.fori_loop(0, 128, zrow, 0)

    def fire(t, buf, tab):
        base = t * KSUP
        for b in range(KSUP):
            pltpu.async_copy(
                tab.at[src_v.at[base + b]], st.at[buf, b], gsem)

    def drain(buf):
        for b in range(KSUP):
            pltpu.make_async_copy(
                yc.at[0, pl.ds(0, CW)], st.at[buf, b], gsem).wait()

    def fire_sc(t, buf):
        base = t * KSUP
        for b in range(KSUP):
            pltpu.async_copy(
                st.at[buf, b], acc.at[dst_v.at[base + b]], ssem, add=True)

    def drain_sc(buf):
        for b in range(KSUP):
            pltpu.make_async_copy(
                yc.at[0, pl.ds(0, CW)], st.at[buf, b], ssem).wait()

    for p in range(2):
        q = 2 * c + p
        tab = yc.at[q]

        def zcp(t, carry):
            pltpu.sync_copy(zb, acc.at[pl.ds(s * RPT + t * 128, 128)])
            return carry
        lax.fori_loop(0, RPT // 128, zcp, 0)
        plsc.subcore_barrier()

        fire(0, 0, tab)

        def super_step(t, carry):
            pp = lax.rem(t, 2)
            drain(pp)

            @pl.when(t >= 1)
            def _():
                drain_sc(1 - pp)

            @pl.when(t + 1 < NSUP)
            def _():
                fire(t + 1, 1 - pp, tab)
            fire_sc(t, pp)
            return carry
        lax.fori_loop(0, NSUP, super_step, 0)
        drain_sc(lax.rem(NSUP - 1, 2))
        plsc.subcore_barrier()

        def out_cp(t, carry):
            pltpu.sync_copy(
                acc.at[pl.ds(s * RPT + t * 128, 128)],
                agg.at[pl.ds(s * RPT + t * 128, 128), pl.ds(q * QW, QW)])
            return carry
        lax.fori_loop(0, RPT // 128, out_cp, 0)
        if p == 0:
            plsc.subcore_barrier()


def _dense_body(a, nd, ns, w1, b1, w2, g):
    h = lax.dot_general(
        a[...], w1[...], (((1,), (0,)), ((), ())),
        precision=lax.Precision.HIGHEST, preferred_element_type=jnp.float32)
    h = h * nd[...] + b1[...]
    h = jnp.maximum(h, 0.0) * ns[...]
    g[...] = jnp.sum(h * w2[...].reshape(1, H), axis=1, keepdims=True)


def _dense(agg, ndst, nsrc, W1, b1, W2):
    return pl.pallas_call(
        _dense_body,
        grid=(GRIDD,),
        in_specs=[
            pl.BlockSpec((BND, F), lambda i: (i, 0)),
            pl.BlockSpec((BND, 1), lambda i: (i, 0)),
            pl.BlockSpec((BND, 1), lambda i: (i, 0)),
            pl.BlockSpec((F, H), lambda i: (0, 0)),
            pl.BlockSpec((1, H), lambda i: (0, 0)),
            pl.BlockSpec((H, 1), lambda i: (0, 0)),
        ],
        out_specs=pl.BlockSpec((BND, 1), lambda i: (i, 0)),
        out_shape=jax.ShapeDtypeStruct((N, 1), jnp.float32),
    )(agg, ndst, nsrc, W1, b1, W2)


@functools.partial(
    pl.kernel,
    out_type=jax.ShapeDtypeStruct((2, NPAD), jnp.float32),
    mesh=_mesh,
    scratch_types=[
        pltpu.VMEM((NCHUNK, CW), jnp.int32),
        pltpu.VMEM((NCHUNK, CW), jnp.int32),
        pltpu.VMEM((NPAD,), jnp.float32),
        pltpu.VMEM((NPAD,), jnp.float32),
        pltpu.VMEM((NPAD // 32,), jnp.float32),
        pltpu.VMEM((NPAD // 32,), jnp.float32),
        pltpu.VMEM((NPAD // 32,), jnp.float32),
        pltpu.VMEM((16,), jnp.float32),
        pltpu.VMEM_SHARED((NSUB, NPAD // 2), jnp.float32),
    ],
    compiler_params=pltpu.CompilerParams(
        use_tc_tiling_on_sc=False, needs_layout_passes=False),
)
def _layer2(g1, s3, d3, ndp, b2h, o2, src_v, dst_v, gtab, accl,
            lbuf, nbuf, obuf, b2v, slots):
    c = lax.axis_index("c")
    s = lax.axis_index("s")
    pltpu.sync_copy(s3.at[s], src_v)
    pltpu.sync_copy(d3.at[s], dst_v)
    pltpu.sync_copy(g1, gtab.at[pl.ds(0, N)])
    pltpu.sync_copy(b2h, b2v)
    _zero_vec(accl, NPAD)

    def edge(j, carry):
        for k in range(CW // 16):
            sl = pl.ds(k * 16, 16)
            si = src_v[j, sl]
            di = dst_v[j, sl]
            vals = plsc.load_gather(gtab, [si])
            plsc.addupdate_scatter(accl, [di], vals)
        return carry
    lax.fori_loop(0, NCHUNK, edge, 0)

    HN = NPAD // 2
    HRPT = NPAD // 32
    b2r = b2v[...]
    for r in range(2):
        pltpu.sync_copy(accl.at[pl.ds(r * HN, HN)], slots.at[s])
        plsc.subcore_barrier()
        off = s * HRPT
        glob = r * HN + off
        pltpu.sync_copy(slots.at[0, pl.ds(off, HRPT)], lbuf)
        for t in range(1, NSUB):
            pltpu.sync_copy(slots.at[t, pl.ds(off, HRPT)], nbuf)

            def accrow(k, carry):
                sl = pl.ds(k * 16, 16)
                lbuf[sl] = lbuf[sl] + nbuf[sl]
                return carry
            lax.fori_loop(0, HRPT // 16, accrow, 0)
        pltpu.sync_copy(ndp.at[pl.ds(glob, HRPT)], nbuf)

        def scale(k, carry):
            sl = pl.ds(k * 16, 16)
            obuf[sl] = lbuf[sl] * nbuf[sl] + b2r
            return carry
        lax.fori_loop(0, HRPT // 16, scale, 0)
        pltpu.sync_copy(obuf, o2.at[c, pl.ds(glob, HRPT)])
        plsc.subcore_barrier()


def kernel(features, edge_index, W1, b1, W2, b2):
    src = edge_index[0].astype(jnp.int32)
    dst = edge_index[1].astype(jnp.int32)
    src3 = src.reshape(NSUB, NCHUNK, CW)
    dst3 = dst.reshape(NSUB, NCHUNK, CW)

    deg2 = _degrees(src3, dst3)
    deg_out = deg2[0, :N, None]
    deg_in = deg2[1, :N, None]

    y, nsrc, ndst = _normalize(features, deg_out, deg_in)
    ycat = y.reshape(N, NQ, QW).transpose(1, 0, 2)

    agg = _aggregate(ycat, src3, dst3)
    g = _dense(agg, ndst, nsrc, W1, b1.reshape(1, H), W2)

    ndp = jnp.concatenate([ndst[:, 0], jnp.zeros((NPAD - N,), jnp.float32)])
    b2h = jnp.broadcast_to(b2, (16,))
    o2 = _layer2(g[:, 0], src3, dst3, ndp, b2h)
    return o2[0, :N].reshape(N, 1)


# speedup vs baseline: 1.8765x; 1.0046x over previous
import functools

import jax
import jax.numpy as jnp
from jax import lax
from jax.experimental import pallas as pl
from jax.experimental.pallas import tpu as pltpu
from jax.experimental.pallas import tpu_sc as plsc

N = 10000
E = 160000
F = 256
H = 512

NSUB = 16
NCHUNK = 125
CW = 80
CWC = 128
NCHC = 80
EP = NSUB * NCHC * CWC
TRASH = 10200
NPAD = 10240
RPT = NPAD // NSUB
BN = 400
GRID = N // BN
BND = 2000
GRIDD = N // BND

_mesh = plsc.VectorSubcoreMesh(core_axis_name="c", subcore_axis_name="s")


def _zero_vec(ref, nwords):
    def body(k, carry):
        ref[pl.ds(k * 16, 16)] = jnp.zeros((16,), jnp.float32)
        return carry
    lax.fori_loop(0, nwords // 16, body, 0)


@functools.partial(
    pl.kernel,
    out_type=jax.ShapeDtypeStruct((2, NPAD), jnp.float32),
    mesh=_mesh,
    scratch_types=[
        pltpu.VMEM((NCHUNK, CW), jnp.int32),
        pltpu.VMEM((CW,), jnp.float32),
        pltpu.VMEM((RPT,), jnp.float32),
        pltpu.VMEM_SHARED((NPAD,), jnp.float32),
    ],
)
def _degrees(src3, dst3, deg2, idx_v, ones_v, buf, acc):
    c = lax.axis_index("c")
    s = lax.axis_index("s")

    @pl.when(c == 0)
    def _():
        pltpu.sync_copy(src3.at[s], idx_v)

    @pl.when(c == 1)
    def _():
        pltpu.sync_copy(dst3.at[s], idx_v)
    for l in range(CW // 16):
        ones_v[pl.ds(l * 16, 16)] = jnp.ones((16,), jnp.float32)
    _zero_vec(buf, RPT)
    pltpu.sync_copy(buf, acc.at[pl.ds(s * RPT, RPT)])
    plsc.subcore_barrier()

    def body(j, carry):
        pltpu.sync_copy(ones_v, acc.at[idx_v.at[j]], add=True)
        return carry
    lax.fori_loop(0, NCHUNK, body, 0)
    plsc.subcore_barrier()
    pltpu.sync_copy(acc.at[pl.ds(s * RPT, RPT)], buf)
    pltpu.sync_copy(buf, deg2.at[c, pl.ds(s * RPT, RPT)])


QW = 64
NQ = F // QW


def _normalize_body(feat, dego, degi, y, nsrc, ndst):
    ns = lax.rsqrt(jnp.maximum(dego[...], 1.0))
    nd = lax.rsqrt(jnp.maximum(degi[...], 1.0))
    y[...] = feat[...] * ns
    nsrc[...] = ns
    ndst[...] = nd


def _normalize(features, deg_out, deg_in):
    return pl.pallas_call(
        _normalize_body,
        grid=(GRID,),
        in_specs=[
            pl.BlockSpec((BN, F), lambda i: (i, 0)),
            pl.BlockSpec((BN, 1), lambda i: (i, 0)),
            pl.BlockSpec((BN, 1), lambda i: (i, 0)),
        ],
        out_specs=[
            pl.BlockSpec((BN, F), lambda i: (i, 0)),
            pl.BlockSpec((BN, 1), lambda i: (i, 0)),
            pl.BlockSpec((BN, 1), lambda i: (i, 0)),
        ],
        out_shape=[
            jax.ShapeDtypeStruct((N, F), jnp.float32),
            jax.ShapeDtypeStruct((N, 1), jnp.float32),
            jax.ShapeDtypeStruct((N, 1), jnp.float32),
        ],
    )(features, deg_out, deg_in)


KSUP = 5

NSUP = NCHUNK // KSUP

@functools.partial(
    pl.kernel,
    out_type=jax.ShapeDtypeStruct((NPAD, F), jnp.float32),
    mesh=_mesh,
    scratch_types=[
        pltpu.VMEM((NCHUNK, CW), jnp.int32),
        pltpu.VMEM((NCHUNK, CW), jnp.int32),
        pltpu.VMEM((2, KSUP, CW, QW), jnp.float32),
        pltpu.VMEM((128, QW), jnp.float32),
        pltpu.VMEM_SHARED((NPAD, QW), jnp.float32),
        pltpu.SemaphoreType.DMA,
        pltpu.SemaphoreType.DMA,
    ],
    compiler_params=pltpu.CompilerParams(use_tc_tiling_on_sc=False),
)
def _aggregate(yc, s3, d3, agg, src_v, dst_v, st, zb, acc, gsem, ssem):
    c = lax.axis_index("c")
    s = lax.axis_index("s")
    pltpu.sync_copy(d3.at[s], dst_v)
    pltpu.sync_copy(s3.at[s], src_v)

    def zrow(r, carry):
        for l in range(QW // 16):
            zb[r, pl.ds(l * 16, 16)] = jnp.zeros((16,), jnp.float32)
        return carry
    lax.fori_loop(0, 128, zrow, 0)

    def fire(t, buf, tab):
        base = t * KSUP
        for b in range(KSUP):
            pltpu.async_copy(
                tab.at[src_v.at[base + b]], st.at[buf, b], gsem)

    def drain(buf):
        for b in range(KSUP):
            pltpu.make_async_copy(
                yc.at[0, pl.ds(0, CW)], st.at[buf, b], gsem).wait()

    def fire_sc(t, buf):
        base = t * KSUP
        for b in range(KSUP):
            pltpu.async_copy(
                st.at[buf, b], acc.at[dst_v.at[base + b]], ssem, add=True)

    def drain_sc(buf):
        for b in range(KSUP):
            pltpu.make_async_copy(
                yc.at[0, pl.ds(0, CW)], st.at[buf, b], ssem).wait()

    def zcp(t, carry):
        pltpu.sync_copy(zb, acc.at[pl.ds(s * RPT + t * 128, 128)])
        return carry

    lax.fori_loop(0, RPT // 128, zcp, 0)
    fire(0, 0, yc.at[2 * c])
    plsc.subcore_barrier()

    for p in range(2):
        q = 2 * c + p
        tab = yc.at[q]

        def super_step(t, carry):
            pp = lax.rem(t, 2)
            drain(pp)

            @pl.when(t >= 1)
            def _():
                drain_sc(1 - pp)

            @pl.when(t + 1 < NSUP)
            def _():
                fire(t + 1, 1 - pp, tab)
            fire_sc(t, pp)
            return carry
        lax.fori_loop(0, NSUP, super_step, 0)
        drain_sc(lax.rem(NSUP - 1, 2))
        if p == 0:
            fire(0, 0, yc.at[2 * c + 1])
        plsc.subcore_barrier()

        def out_cp(t, carry):
            pltpu.sync_copy(
                acc.at[pl.ds(s * RPT + t * 128, 128)],
                agg.at[pl.ds(s * RPT + t * 128, 128), pl.ds(q * QW, QW)])
            return carry
        lax.fori_loop(0, RPT // 128, out_cp, 0)
        if p == 0:
            lax.fori_loop(0, RPT // 128, zcp, 0)
            plsc.subcore_barrier()


def _dense_body(a, nd, ns, w1, b1, w2, g):
    h = lax.dot_general(
        a[...], w1[...], (((1,), (0,)), ((), ())),
        precision=lax.Precision.HIGHEST, preferred_element_type=jnp.float32)
    h = h * nd[...] + b1[...]
    h = jnp.maximum(h, 0.0) * ns[...]
    g[...] = jnp.sum(h * w2[...].reshape(1, H), axis=1, keepdims=True)


def _dense(agg, ndst, nsrc, W1, b1, W2):
    return pl.pallas_call(
        _dense_body,
        grid=(GRIDD,),
        in_specs=[
            pl.BlockSpec((BND, F), lambda i: (i, 0)),
            pl.BlockSpec((BND, 1), lambda i: (i, 0)),
            pl.BlockSpec((BND, 1), lambda i: (i, 0)),
            pl.BlockSpec((F, H), lambda i: (0, 0)),
            pl.BlockSpec((1, H), lambda i: (0, 0)),
            pl.BlockSpec((H, 1), lambda i: (0, 0)),
        ],
        out_specs=pl.BlockSpec((BND, 1), lambda i: (i, 0)),
        out_shape=jax.ShapeDtypeStruct((N, 1), jnp.float32),
    )(agg, ndst, nsrc, W1, b1, W2)


@functools.partial(
    pl.kernel,
    out_type=jax.ShapeDtypeStruct((2, NPAD), jnp.float32),
    mesh=_mesh,
    scratch_types=[
        pltpu.VMEM((NCHUNK, CW), jnp.int32),
        pltpu.VMEM((NCHUNK, CW), jnp.int32),
        pltpu.VMEM((NPAD,), jnp.float32),
        pltpu.VMEM((NPAD,), jnp.float32),
        pltpu.VMEM((NPAD // 32,), jnp.float32),
        pltpu.VMEM((NPAD // 32,), jnp.float32),
        pltpu.VMEM((NPAD // 32,), jnp.float32),
        pltpu.VMEM((16,), jnp.float32),
        pltpu.VMEM_SHARED((NSUB, NPAD // 2), jnp.float32),
    ],
    compiler_params=pltpu.CompilerParams(
        use_tc_tiling_on_sc=False, needs_layout_passes=False),
)
def _layer2(g1, s3, d3, ndp, b2h, o2, src_v, dst_v, gtab, accl,
            lbuf, nbuf, obuf, b2v, slots):
    c = lax.axis_index("c")
    s = lax.axis_index("s")
    pltpu.sync_copy(s3.at[s], src_v)
    pltpu.sync_copy(d3.at[s], dst_v)
    pltpu.sync_copy(g1, gtab.at[pl.ds(0, N)])
    pltpu.sync_copy(b2h, b2v)
    _zero_vec(accl, NPAD)

    def edge(j, carry):
        for k in range(CW // 16):
            sl = pl.ds(k * 16, 16)
            si = src_v[j, sl]
            di = dst_v[j, sl]
            vals = plsc.load_gather(gtab, [si])
            plsc.addupdate_scatter(accl, [di], vals)
        return carry
    lax.fori_loop(0, NCHUNK, edge, 0)

    HN = NPAD // 2
    HRPT = NPAD // 32
    b2r = b2v[...]
    for r in range(2):
        pltpu.sync_copy(accl.at[pl.ds(r * HN, HN)], slots.at[s])
        plsc.subcore_barrier()
        off = s * HRPT
        glob = r * HN + off
        pltpu.sync_copy(slots.at[0, pl.ds(off, HRPT)], lbuf)
        for t in range(1, NSUB):
            pltpu.sync_copy(slots.at[t, pl.ds(off, HRPT)], nbuf)

            def accrow(k, carry):
                sl = pl.ds(k * 16, 16)
                lbuf[sl] = lbuf[sl] + nbuf[sl]
                return carry
            lax.fori_loop(0, HRPT // 16, accrow, 0)
        pltpu.sync_copy(ndp.at[pl.ds(glob, HRPT)], nbuf)

        def scale(k, carry):
            sl = pl.ds(k * 16, 16)
            obuf[sl] = lbuf[sl] * nbuf[sl] + b2r
            return carry
        lax.fori_loop(0, HRPT // 16, scale, 0)
        pltpu.sync_copy(obuf, o2.at[c, pl.ds(glob, HRPT)])
        plsc.subcore_barrier()


def kernel(features, edge_index, W1, b1, W2, b2):
    src = edge_index[0].astype(jnp.int32)
    dst = edge_index[1].astype(jnp.int32)
    src3 = src.reshape(NSUB, NCHUNK, CW)
    dst3 = dst.reshape(NSUB, NCHUNK, CW)

    deg2 = _degrees(src3, dst3)
    deg_out = deg2[0, :N, None]
    deg_in = deg2[1, :N, None]

    y, nsrc, ndst = _normalize(features, deg_out, deg_in)
    ycat = y.reshape(N, NQ, QW).transpose(1, 0, 2)

    agg = _aggregate(ycat, src3, dst3)
    g = _dense(agg, ndst, nsrc, W1, b1.reshape(1, H), W2)

    ndp = jnp.concatenate([ndst[:, 0], jnp.zeros((NPAD - N,), jnp.float32)])
    b2h = jnp.broadcast_to(b2, (16,))
    o2 = _layer2(g[:, 0], src3, dst3, ndp, b2h)
    return o2[0, :N].reshape(N, 1)


# speedup vs baseline: 1.9507x; 1.0395x over previous
import functools

import jax
import jax.numpy as jnp
from jax import lax
from jax.experimental import pallas as pl
from jax.experimental.pallas import tpu as pltpu
from jax.experimental.pallas import tpu_sc as plsc

N = 10000
E = 160000
F = 256
H = 512

NSUB = 16
NCHUNK = 125
CW = 80
CWC = 128
NCHC = 80
EP = NSUB * NCHC * CWC
TRASH = 10200
NPAD = 10240
RPT = NPAD // NSUB
BN = 400
GRID = N // BN
BND = 2000
GRIDD = N // BND

_mesh = plsc.VectorSubcoreMesh(core_axis_name="c", subcore_axis_name="s")


def _zero_vec(ref, nwords):
    def body(k, carry):
        ref[pl.ds(k * 16, 16)] = jnp.zeros((16,), jnp.float32)
        return carry
    lax.fori_loop(0, nwords // 16, body, 0)


@functools.partial(
    pl.kernel,
    out_type=jax.ShapeDtypeStruct((2, NPAD), jnp.float32),
    mesh=_mesh,
    scratch_types=[
        pltpu.VMEM((NCHUNK, CW), jnp.int32),
        pltpu.VMEM((CW,), jnp.float32),
        pltpu.VMEM((RPT,), jnp.float32),
        pltpu.VMEM_SHARED((NPAD,), jnp.float32),
    ],
)
def _degrees(src3, dst3, deg2, idx_v, ones_v, buf, acc):
    c = lax.axis_index("c")
    s = lax.axis_index("s")

    @pl.when(c == 0)
    def _():
        pltpu.sync_copy(src3.at[s], idx_v)

    @pl.when(c == 1)
    def _():
        pltpu.sync_copy(dst3.at[s], idx_v)
    for l in range(CW // 16):
        ones_v[pl.ds(l * 16, 16)] = jnp.ones((16,), jnp.float32)
    _zero_vec(buf, RPT)
    pltpu.sync_copy(buf, acc.at[pl.ds(s * RPT, RPT)])
    plsc.subcore_barrier()

    def body(j, carry):
        pltpu.sync_copy(ones_v, acc.at[idx_v.at[j]], add=True)
        return carry
    lax.fori_loop(0, NCHUNK, body, 0)
    plsc.subcore_barrier()
    pltpu.sync_copy(acc.at[pl.ds(s * RPT, RPT)], buf)
    pltpu.sync_copy(buf, deg2.at[c, pl.ds(s * RPT, RPT)])


QW = 64
NQ = F // QW


def _normalize_body(feat, dego, degi, y, nsrc, ndst):
    ns = lax.rsqrt(jnp.maximum(dego[...], 1.0))
    nd = lax.rsqrt(jnp.maximum(degi[...], 1.0))
    y[...] = feat[...] * ns
    nsrc[...] = ns
    ndst[...] = nd


def _normalize(features, deg_out, deg_in):
    return pl.pallas_call(
        _normalize_body,
        grid=(GRID,),
        in_specs=[
            pl.BlockSpec((BN, F), lambda i: (i, 0)),
            pl.BlockSpec((BN, 1), lambda i: (i, 0)),
            pl.BlockSpec((BN, 1), lambda i: (i, 0)),
        ],
        out_specs=[
            pl.BlockSpec((BN, F), lambda i: (i, 0)),
            pl.BlockSpec((BN, 1), lambda i: (i, 0)),
            pl.BlockSpec((BN, 1), lambda i: (i, 0)),
        ],
        out_shape=[
            jax.ShapeDtypeStruct((N, F), jnp.float32),
            jax.ShapeDtypeStruct((N, 1), jnp.float32),
            jax.ShapeDtypeStruct((N, 1), jnp.float32),
        ],
    )(features, deg_out, deg_in)


KSUP = 5

NSUP = NCHUNK // KSUP

@functools.partial(
    pl.kernel,
    out_type=jax.ShapeDtypeStruct((NPAD, F), jnp.float32),
    mesh=_mesh,
    scratch_types=[
        pltpu.VMEM((NCHUNK, CW), jnp.int32),
        pltpu.VMEM((NCHUNK, CW), jnp.int32),
        pltpu.VMEM((2, KSUP, CW, QW), jnp.float32),
        pltpu.VMEM((128, QW), jnp.float32),
        pltpu.VMEM_SHARED((NPAD, QW), jnp.float32),
        pltpu.SemaphoreType.DMA,
        pltpu.SemaphoreType.DMA,
    ],
    compiler_params=pltpu.CompilerParams(use_tc_tiling_on_sc=False),
)
def _aggregate(yc, s3, d3, agg, src_v, dst_v, st, zb, acc, gsem, ssem):
    c = lax.axis_index("c")
    s = lax.axis_index("s")
    pltpu.sync_copy(d3.at[s], dst_v)
    pltpu.sync_copy(s3.at[s], src_v)

    def zrow(r, carry):
        for l in range(QW // 16):
            zb[r, pl.ds(l * 16, 16)] = jnp.zeros((16,), jnp.float32)
        return carry
    lax.fori_loop(0, 128, zrow, 0)

    def fire(t, buf, tab):
        base = t * KSUP
        for b in range(KSUP):
            pltpu.async_copy(
                tab.at[src_v.at[base + b]], st.at[buf, b], gsem)

    def drain(buf):
        for b in range(KSUP):
            pltpu.make_async_copy(
                yc.at[0, pl.ds(0, CW)], st.at[buf, b], gsem).wait()

    def fire_sc(t, buf):
        base = t * KSUP
        for b in range(KSUP):
            pltpu.async_copy(
                st.at[buf, b], acc.at[dst_v.at[base + b]], ssem, add=True)

    def drain_sc(buf):
        for b in range(KSUP):
            pltpu.make_async_copy(
                yc.at[0, pl.ds(0, CW)], st.at[buf, b], ssem).wait()

    def zcp(t, carry):
        pltpu.sync_copy(zb, acc.at[pl.ds(s * RPT + t * 128, 128)])
        return carry

    lax.fori_loop(0, RPT // 128, zcp, 0)
    fire(0, 0, yc.at[2 * c])
    plsc.subcore_barrier()

    for p in range(2):
        q = 2 * c + p
        tab = yc.at[q]

        def super_step(t, carry):
            pp = lax.rem(t, 2)
            drain(pp)

            @pl.when(t >= 1)
            def _():
                drain_sc(1 - pp)

            @pl.when(t + 1 < NSUP)
            def _():
                fire(t + 1, 1 - pp, tab)
            fire_sc(t, pp)
            return carry
        lax.fori_loop(0, NSUP, super_step, 0)
        drain_sc(lax.rem(NSUP - 1, 2))
        if p == 0:
            fire(0, 0, yc.at[2 * c + 1])
        plsc.subcore_barrier()

        def out_cp(t, carry):
            pltpu.sync_copy(
                acc.at[pl.ds(s * RPT + t * 128, 128)],
                agg.at[pl.ds(s * RPT + t * 128, 128), pl.ds(q * QW, QW)])
            return carry
        lax.fori_loop(0, RPT // 128, out_cp, 0)
        if p == 0:
            lax.fori_loop(0, RPT // 128, zcp, 0)
            plsc.subcore_barrier()


def _dense_body(a, nd, ns, w1, b1, w2, g):
    h = lax.dot_general(
        a[...], w1[...], (((1,), (0,)), ((), ())),
        precision=lax.Precision.DEFAULT, preferred_element_type=jnp.float32)
    h = h * nd[...] + b1[...]
    h = jnp.maximum(h, 0.0) * ns[...]
    g[...] = jnp.sum(h * w2[...].reshape(1, H), axis=1, keepdims=True)


def _dense(agg, ndst, nsrc, W1, b1, W2):
    return pl.pallas_call(
        _dense_body,
        grid=(GRIDD,),
        in_specs=[
            pl.BlockSpec((BND, F), lambda i: (i, 0)),
            pl.BlockSpec((BND, 1), lambda i: (i, 0)),
            pl.BlockSpec((BND, 1), lambda i: (i, 0)),
            pl.BlockSpec((F, H), lambda i: (0, 0)),
            pl.BlockSpec((1, H), lambda i: (0, 0)),
            pl.BlockSpec((H, 1), lambda i: (0, 0)),
        ],
        out_specs=pl.BlockSpec((BND, 1), lambda i: (i, 0)),
        out_shape=jax.ShapeDtypeStruct((N, 1), jnp.float32),
    )(agg, ndst, nsrc, W1, b1, W2)


@functools.partial(
    pl.kernel,
    out_type=jax.ShapeDtypeStruct((2, NPAD), jnp.float32),
    mesh=_mesh,
    scratch_types=[
        pltpu.VMEM((NCHUNK, CW), jnp.int32),
        pltpu.VMEM((NCHUNK, CW), jnp.int32),
        pltpu.VMEM((NPAD,), jnp.float32),
        pltpu.VMEM((NPAD,), jnp.float32),
        pltpu.VMEM((NPAD // 32,), jnp.float32),
        pltpu.VMEM((NPAD // 32,), jnp.float32),
        pltpu.VMEM((NPAD // 32,), jnp.float32),
        pltpu.VMEM((16,), jnp.float32),
        pltpu.VMEM_SHARED((NSUB, NPAD // 2), jnp.float32),
    ],
    compiler_params=pltpu.CompilerParams(
        use_tc_tiling_on_sc=False, needs_layout_passes=False),
)
def _layer2(g1, s3, d3, ndp, b2h, o2, src_v, dst_v, gtab, accl,
            lbuf, nbuf, obuf, b2v, slots):
    c = lax.axis_index("c")
    s = lax.axis_index("s")
    pltpu.sync_copy(s3.at[s], src_v)
    pltpu.sync_copy(d3.at[s], dst_v)
    pltpu.sync_copy(g1, gtab.at[pl.ds(0, N)])
    pltpu.sync_copy(b2h, b2v)
    _zero_vec(accl, NPAD)

    def edge(j, carry):
        for k in range(CW // 16):
            sl = pl.ds(k * 16, 16)
            si = src_v[j, sl]
            di = dst_v[j, sl]
            vals = plsc.load_gather(gtab, [si])
            plsc.addupdate_scatter(accl, [di], vals)
        return carry
    lax.fori_loop(0, NCHUNK, edge, 0)

    HN = NPAD // 2
    HRPT = NPAD // 32
    b2r = b2v[...]
    for r in range(2):
        pltpu.sync_copy(accl.at[pl.ds(r * HN, HN)], slots.at[s])
        plsc.subcore_barrier()
        off = s * HRPT
        glob = r * HN + off
        pltpu.sync_copy(slots.at[0, pl.ds(off, HRPT)], lbuf)
        for t in range(1, NSUB):
            pltpu.sync_copy(slots.at[t, pl.ds(off, HRPT)], nbuf)

            def accrow(k, carry):
                sl = pl.ds(k * 16, 16)
                lbuf[sl] = lbuf[sl] + nbuf[sl]
                return carry
            lax.fori_loop(0, HRPT // 16, accrow, 0)
        pltpu.sync_copy(ndp.at[pl.ds(glob, HRPT)], nbuf)

        def scale(k, carry):
            sl = pl.ds(k * 16, 16)
            obuf[sl] = lbuf[sl] * nbuf[sl] + b2r
            return carry
        lax.fori_loop(0, HRPT // 16, scale, 0)
        pltpu.sync_copy(obuf, o2.at[c, pl.ds(glob, HRPT)])
        plsc.subcore_barrier()


def kernel(features, edge_index, W1, b1, W2, b2):
    src = edge_index[0].astype(jnp.int32)
    dst = edge_index[1].astype(jnp.int32)
    src3 = src.reshape(NSUB, NCHUNK, CW)
    dst3 = dst.reshape(NSUB, NCHUNK, CW)

    deg2 = _degrees(src3, dst3)
    deg_out = deg2[0, :N, None]
    deg_in = deg2[1, :N, None]

    y, nsrc, ndst = _normalize(features, deg_out, deg_in)
    ycat = y.reshape(N, NQ, QW).transpose(1, 0, 2)

    agg = _aggregate(ycat, src3, dst3)
    g = _dense(agg, ndst, nsrc, W1, b1.reshape(1, H), W2)

    ndp = jnp.concatenate([ndst[:, 0], jnp.zeros((NPAD - N,), jnp.float32)])
    b2h = jnp.broadcast_to(b2, (16,))
    o2 = _layer2(g[:, 0], src3, dst3, ndp, b2h)
    return o2[0, :N].reshape(N, 1)


# speedup vs baseline: 1.9916x; 1.0210x over previous
import functools

import jax
import jax.numpy as jnp
from jax import lax
from jax.experimental import pallas as pl
from jax.experimental.pallas import tpu as pltpu
from jax.experimental.pallas import tpu_sc as plsc

N = 10000
E = 160000
F = 256
H = 512

NSUB = 16
NCHUNK = 125
CW = 80
CWC = 128
NCHC = 80
EP = NSUB * NCHC * CWC
TRASH = 10200
NPAD = 10240
RPT = NPAD // NSUB
BN = 400
GRID = N // BN
BND = 2000
GRIDD = N // BND

_mesh = plsc.VectorSubcoreMesh(core_axis_name="c", subcore_axis_name="s")


def _zero_vec(ref, nwords):
    def body(k, carry):
        ref[pl.ds(k * 16, 16)] = jnp.zeros((16,), jnp.float32)
        return carry
    lax.fori_loop(0, nwords // 16, body, 0)


@functools.partial(
    pl.kernel,
    out_type=jax.ShapeDtypeStruct((2, NPAD), jnp.float32),
    mesh=_mesh,
    scratch_types=[
        pltpu.VMEM((NCHUNK, CW), jnp.int32),
        pltpu.VMEM((CW,), jnp.float32),
        pltpu.VMEM((RPT,), jnp.float32),
        pltpu.VMEM_SHARED((NPAD,), jnp.float32),
        pltpu.SemaphoreType.DMA,
    ],
)
def _degrees(src3, dst3, deg2, idx_v, ones_v, buf, acc, sem):
    c = lax.axis_index("c")
    s = lax.axis_index("s")

    @pl.when(c == 0)
    def _():
        pltpu.sync_copy(src3.at[s], idx_v)

    @pl.when(c == 1)
    def _():
        pltpu.sync_copy(dst3.at[s], idx_v)
    for l in range(CW // 16):
        ones_v[pl.ds(l * 16, 16)] = jnp.ones((16,), jnp.float32)
    _zero_vec(buf, RPT)
    pltpu.sync_copy(buf, acc.at[pl.ds(s * RPT, RPT)])
    plsc.subcore_barrier()

    def body(j, carry):
        for b in range(5):
            pltpu.async_copy(ones_v, acc.at[idx_v.at[5 * j + b]], sem,
                             add=True)
        for b in range(5):
            pltpu.make_async_copy(ones_v, acc.at[idx_v.at[0]], sem).wait()
        return carry
    lax.fori_loop(0, NCHUNK // 5, body, 0)
    plsc.subcore_barrier()
    pltpu.sync_copy(acc.at[pl.ds(s * RPT, RPT)], buf)
    pltpu.sync_copy(buf, deg2.at[c, pl.ds(s * RPT, RPT)])


QW = 64
NQ = F // QW


def _normalize_body(feat, dego, degi, y, nsrc, ndst):
    ns = lax.rsqrt(jnp.maximum(dego[...], 1.0))
    nd = lax.rsqrt(jnp.maximum(degi[...], 1.0))
    y[...] = feat[...] * ns
    nsrc[...] = ns
    ndst[...] = nd


def _normalize(features, deg_out, deg_in):
    return pl.pallas_call(
        _normalize_body,
        grid=(GRID,),
        in_specs=[
            pl.BlockSpec((BN, F), lambda i: (i, 0)),
            pl.BlockSpec((BN, 1), lambda i: (i, 0)),
            pl.BlockSpec((BN, 1), lambda i: (i, 0)),
        ],
        out_specs=[
            pl.BlockSpec((BN, F), lambda i: (i, 0)),
            pl.BlockSpec((BN, 1), lambda i: (i, 0)),
            pl.BlockSpec((BN, 1), lambda i: (i, 0)),
        ],
        out_shape=[
            jax.ShapeDtypeStruct((N, F), jnp.float32),
            jax.ShapeDtypeStruct((N, 1), jnp.float32),
            jax.ShapeDtypeStruct((N, 1), jnp.float32),
        ],
    )(features, deg_out, deg_in)


KSUP = 5

NSUP = NCHUNK // KSUP

@functools.partial(
    pl.kernel,
    out_type=jax.ShapeDtypeStruct((NPAD, F), jnp.float32),
    mesh=_mesh,
    scratch_types=[
        pltpu.VMEM((NCHUNK, CW), jnp.int32),
        pltpu.VMEM((NCHUNK, CW), jnp.int32),
        pltpu.VMEM((2, KSUP, CW, QW), jnp.float32),
        pltpu.VMEM((128, QW), jnp.float32),
        pltpu.VMEM_SHARED((NPAD, QW), jnp.float32),
        pltpu.SemaphoreType.DMA,
        pltpu.SemaphoreType.DMA,
    ],
    compiler_params=pltpu.CompilerParams(use_tc_tiling_on_sc=False),
)
def _aggregate(yc, s3, d3, agg, src_v, dst_v, st, zb, acc, gsem, ssem):
    c = lax.axis_index("c")
    s = lax.axis_index("s")
    pltpu.sync_copy(d3.at[s], dst_v)
    pltpu.sync_copy(s3.at[s], src_v)

    def zrow(r, carry):
        for l in range(QW // 16):
            zb[r, pl.ds(l * 16, 16)] = jnp.zeros((16,), jnp.float32)
        return carry
    lax.fori_loop(0, 128, zrow, 0)

    def fire(t, buf, tab):
        base = t * KSUP
        for b in range(KSUP):
            pltpu.async_copy(
                tab.at[src_v.at[base + b]], st.at[buf, b], gsem)

    def drain(buf):
        for b in range(KSUP):
            pltpu.make_async_copy(
                yc.at[0, pl.ds(0, CW)], st.at[buf, b], gsem).wait()

    def fire_sc(t, buf):
        base = t * KSUP
        for b in range(KSUP):
            pltpu.async_copy(
                st.at[buf, b], acc.at[dst_v.at[base + b]], ssem, add=True)

    def drain_sc(buf):
        for b in range(KSUP):
            pltpu.make_async_copy(
                yc.at[0, pl.ds(0, CW)], st.at[buf, b], ssem).wait()

    def zcp(t, carry):
        pltpu.sync_copy(zb, acc.at[pl.ds(s * RPT + t * 128, 128)])
        return carry

    lax.fori_loop(0, RPT // 128, zcp, 0)
    fire(0, 0, yc.at[2 * c])
    plsc.subcore_barrier()

    for p in range(2):
        q = 2 * c + p
        tab = yc.at[q]

        def super_step(t, carry):
            pp = lax.rem(t, 2)
            drain(pp)

            @pl.when(t >= 1)
            def _():
                drain_sc(1 - pp)

            @pl.when(t + 1 < NSUP)
            def _():
                fire(t + 1, 1 - pp, tab)
            fire_sc(t, pp)
            return carry
        lax.fori_loop(0, NSUP, super_step, 0)
        drain_sc(lax.rem(NSUP - 1, 2))
        if p == 0:
            fire(0, 0, yc.at[2 * c + 1])
        plsc.subcore_barrier()

        def out_cp(t, carry):
            pltpu.sync_copy(
                acc.at[pl.ds(s * RPT + t * 128, 128)],
                agg.at[pl.ds(s * RPT + t * 128, 128), pl.ds(q * QW, QW)])
            return carry
        lax.fori_loop(0, RPT // 128, out_cp, 0)
        if p == 0:
            lax.fori_loop(0, RPT // 128, zcp, 0)
            plsc.subcore_barrier()


def _dense_body(a, nd, ns, w1, b1, w2, g):
    h = lax.dot_general(
        a[...], w1[...], (((1,), (0,)), ((), ())),
        precision=lax.Precision.DEFAULT, preferred_element_type=jnp.float32)
    h = h * nd[...] + b1[...]
    h = jnp.maximum(h, 0.0) * ns[...]
    g[...] = jnp.sum(h * w2[...].reshape(1, H), axis=1, keepdims=True)


def _dense(agg, ndst, nsrc, W1, b1, W2):
    return pl.pallas_call(
        _dense_body,
        grid=(GRIDD,),
        in_specs=[
            pl.BlockSpec((BND, F), lambda i: (i, 0)),
            pl.BlockSpec((BND, 1), lambda i: (i, 0)),
            pl.BlockSpec((BND, 1), lambda i: (i, 0)),
            pl.BlockSpec((F, H), lambda i: (0, 0)),
            pl.BlockSpec((1, H), lambda i: (0, 0)),
            pl.BlockSpec((H, 1), lambda i: (0, 0)),
        ],
        out_specs=pl.BlockSpec((BND, 1), lambda i: (i, 0)),
        out_shape=jax.ShapeDtypeStruct((N, 1), jnp.float32),
    )(agg, ndst, nsrc, W1, b1, W2)


@functools.partial(
    pl.kernel,
    out_type=jax.ShapeDtypeStruct((2, NPAD), jnp.float32),
    mesh=_mesh,
    scratch_types=[
        pltpu.VMEM((NCHUNK, CW), jnp.int32),
        pltpu.VMEM((NCHUNK, CW), jnp.int32),
        pltpu.VMEM((NPAD,), jnp.float32),
        pltpu.VMEM((NPAD,), jnp.float32),
        pltpu.VMEM((NPAD // 32,), jnp.float32),
        pltpu.VMEM((NPAD // 32,), jnp.float32),
        pltpu.VMEM((NPAD // 32,), jnp.float32),
        pltpu.VMEM((16,), jnp.float32),
        pltpu.VMEM_SHARED((NSUB, NPAD // 2), jnp.float32),
    ],
    compiler_params=pltpu.CompilerParams(
        use_tc_tiling_on_sc=False, needs_layout_passes=False),
)
def _layer2(g1, s3, d3, ndp, b2h, o2, src_v, dst_v, gtab, accl,
            lbuf, nbuf, obuf, b2v, slots):
    c = lax.axis_index("c")
    s = lax.axis_index("s")
    pltpu.sync_copy(s3.at[s], src_v)
    pltpu.sync_copy(d3.at[s], dst_v)
    pltpu.sync_copy(g1, gtab.at[pl.ds(0, N)])
    pltpu.sync_copy(b2h, b2v)
    _zero_vec(accl, NPAD)

    def edge(j, carry):
        for k in range(CW // 16):
            sl = pl.ds(k * 16, 16)
            si = src_v[j, sl]
            di = dst_v[j, sl]
            vals = plsc.load_gather(gtab, [si])
            plsc.addupdate_scatter(accl, [di], vals)
        return carry
    lax.fori_loop(0, NCHUNK, edge, 0)

    HN = NPAD // 2
    HRPT = NPAD // 32
    b2r = b2v[...]
    for r in range(2):
        pltpu.sync_copy(accl.at[pl.ds(r * HN, HN)], slots.at[s])
        plsc.subcore_barrier()
        off = s * HRPT
        glob = r * HN + off
        pltpu.sync_copy(slots.at[0, pl.ds(off, HRPT)], lbuf)
        for t in range(1, NSUB):
            pltpu.sync_copy(slots.at[t, pl.ds(off, HRPT)], nbuf)

            def accrow(k, carry):
                sl = pl.ds(k * 16, 16)
                lbuf[sl] = lbuf[sl] + nbuf[sl]
                return carry
            lax.fori_loop(0, HRPT // 16, accrow, 0)
        pltpu.sync_copy(ndp.at[pl.ds(glob, HRPT)], nbuf)

        def scale(k, carry):
            sl = pl.ds(k * 16, 16)
            obuf[sl] = lbuf[sl] * nbuf[sl] + b2r
            return carry
        lax.fori_loop(0, HRPT // 16, scale, 0)
        pltpu.sync_copy(obuf, o2.at[c, pl.ds(glob, HRPT)])
        plsc.subcore_barrier()


def kernel(features, edge_index, W1, b1, W2, b2):
    src = edge_index[0].astype(jnp.int32)
    dst = edge_index[1].astype(jnp.int32)
    src3 = src.reshape(NSUB, NCHUNK, CW)
    dst3 = dst.reshape(NSUB, NCHUNK, CW)

    deg2 = _degrees(src3, dst3)
    deg_out = deg2[0, :N, None]
    deg_in = deg2[1, :N, None]

    y, nsrc, ndst = _normalize(features, deg_out, deg_in)
    ycat = y.reshape(N, NQ, QW).transpose(1, 0, 2)

    agg = _aggregate(ycat, src3, dst3)
    g = _dense(agg, ndst, nsrc, W1, b1.reshape(1, H), W2)

    ndp = jnp.concatenate([ndst[:, 0], jnp.zeros((NPAD - N,), jnp.float32)])
    b2h = jnp.broadcast_to(b2, (16,))
    o2 = _layer2(g[:, 0], src3, dst3, ndp, b2h)
    return o2[0, :N].reshape(N, 1)
